# baseline (speedup 1.0000x reference)
import jax
import jax.numpy as jnp
from jax import lax
from jax.experimental import pallas as pl
from jax.experimental.pallas import tpu as pltpu

_E = 8
_B = 4096
_C = 2
_NN = 100000
_NE = 1600000
_TRAIN_AFTER = 10
_ALPHA = 1.5
_W_CE, _W_REG, _W_SEM, _W_STR, _W_DIV, _W_LOAD = 1.0, 0.5, 0.5, 0.5, 0.1, 0.01

_CHUNK = 400000
_NSTEP = _NE // _CHUNK
_ITERS_PER_STEP = 12
_G1, _G2 = 8, 512


def _sqp(z):
    zc = jnp.maximum(z, 0.0)
    return zc * zc


def _eyef(k):
    return (lax.broadcasted_iota(jnp.int32, (k, k), 0)
            == lax.broadcasted_iota(jnp.int32, (k, k), 1)).astype(jnp.float32)


def _offdiag_mean_from_gram(G):
    K = G.shape[0]
    eyef = _eyef(K)
    diag_row = jnp.sum(G * eyef, axis=0, keepdims=True)
    ninv_row = 1.0 / jnp.maximum(jnp.sqrt(diag_row), 1e-12)
    ninv_col = jnp.sum(eyef * ninv_row, axis=1, keepdims=True)
    S = G * ninv_col * ninv_row
    full = jnp.sum(S)
    diag = jnp.sum(S * eyef)
    return (full - diag) / (K * (K - 1))


def _body(gate_ref, el0_ref, el1_ref, aux_ref, node_ref, edge_ref,
          agg_ref, total_ref, acc_ref, xs_ref, st_ref, sm_ref):
    i = pl.program_id(0)

    @pl.when(i == 0)
    def _init_acc():
        acc_ref[:, :] = jnp.zeros((_E, _E), jnp.float32)

    x = edge_ref[:, :]
    acc_ref[:, :] += lax.dot_general(
        x, x, (((1,), (1,)), ((), ())), preferred_element_type=jnp.float32)

    @pl.when(i == 0)
    def _init():
        nm = node_ref[:, :]
        Gn = lax.dot_general(nm, nm, (((1,), (1,)), ((), ())),
                             preferred_element_type=jnp.float32)
        sm_ref[0] = _offdiag_mean_from_gram(Gn)

        gate = gate_ref[:, :, :]
        flag = aux_ref[1:2, 0:1, 24:25]
        uniform = jnp.full((_E, _G1, _G2), 1.0 / _E, jnp.float32)
        gw0 = jnp.where(flag > 0.0, uniform, gate)
        Xs = gw0 * (_ALPHA - 1.0)
        xs_ref[:, :, :] = Xs
        max_val = jnp.max(Xs, axis=0, keepdims=True)
        tau_lo = max_val - 1.0
        tau_hi = max_val - (1.0 / _E) ** (_ALPHA - 1.0)
        f_lo = jnp.sum(_sqp(Xs - tau_lo), axis=0, keepdims=True) - 1.0
        st_ref[0:1] = tau_lo
        st_ref[1:2] = tau_hi - tau_lo
        st_ref[2:3] = tau_lo
        st_ref[3:4] = f_lo

    @pl.when(i > 0)
    def _bisect():
        Xs = xs_ref[:, :, :]
        tau_lo = st_ref[0:1]
        dm = st_ref[1:2]
        f_lo = st_ref[3:4]
        tau_m = tau_lo
        for _ in range(_ITERS_PER_STEP):
            dm = dm / 2.0
            tau_m = tau_lo + dm
            p_m = _sqp(Xs - tau_m)
            f_m = jnp.sum(p_m, axis=0, keepdims=True) - 1.0
            tau_lo = jnp.where((f_m * f_lo) >= 0, tau_m, tau_lo)
        st_ref[0:1] = tau_lo
        st_ref[1:2] = dm
        st_ref[2:3] = tau_m

    @pl.when(i == _NSTEP - 1)
    def _final():
        Xs = xs_ref[:, :, :]
        p_m = _sqp(Xs - st_ref[2:3])
        gw = p_m / jnp.sum(p_m, axis=0, keepdims=True)

        agg0 = jnp.sum(el0_ref[:, :, :] * gw, axis=0, keepdims=True)
        agg1 = jnp.sum(el1_ref[:, :, :] * gw, axis=0, keepdims=True)
        agg_ref[0:1, :] = jnp.reshape(agg0, (1, _B))
        agg_ref[1:2, :] = jnp.reshape(agg1, (1, _B))

        yf = aux_ref[0:1]
        c1 = jnp.sum(yf)
        c0 = jnp.float32(_B) - c1
        c0 = jnp.where(c0 == 0.0, 1.0, c0)
        c1 = jnp.where(c1 == 0.0, 1.0, c1)
        w0 = 1.0 / c0
        w1 = 1.0 / c1
        wsum = w0 + w1
        w0 = w0 / wsum
        w1 = w1 / wsum
        m = jnp.maximum(agg0, agg1)
        lse = m + jnp.log(jnp.exp(agg0 - m) + jnp.exp(agg1 - m))
        logp0 = agg0 - lse
        logp1 = agg1 - lse
        is0 = yf == 0.0
        nll = -jnp.where(is0, logp0, logp1)
        wi = jnp.where(is0, w0, w1)
        ce = jnp.sum(wi * nll) / jnp.sum(wi)

        wf_col = jnp.reshape(gw[:, 0:1, 0:1], (_E, 1))
        wf_row = jnp.sum(_eyef(_E) * wf_col, axis=0, keepdims=True)
        wf3 = jnp.reshape(wf_row, (1, 1, _E))
        reg = jnp.sum(aux_ref[1:2, 0:1, 0:8] * wf3)
        sem = jnp.sum(aux_ref[1:2, 0:1, 8:16] * wf3)
        strv = jnp.sum(aux_ref[1:2, 0:1, 16:24] * wf3)

        s2 = jnp.sum(gw, axis=2, keepdims=True)
        avg = jnp.sum(s2, axis=1, keepdims=True) / jnp.float32(_B)
        u = 1.0 / _E
        load = jnp.sum(u * (jnp.log(jnp.full((_E, 1, 1), u, jnp.float32))
                            - jnp.log(avg + 1e-8))) / _E

        off_edge = _offdiag_mean_from_gram(acc_ref[:, :])
        div = (sm_ref[0] + off_edge) / 2.0
        total = (_W_CE * ce + _W_REG * reg + _W_SEM * sem + _W_STR * strv
                 + _W_DIV * div + _W_LOAD * load)
        total_ref[0:1, 0:1] = jnp.reshape(total, (1, 1))


def kernel(gate_logits, expert_logits, node_masks, edge_masks,
           loss_reg, loss_sem, loss_str, y, epoch):
    flag = (jnp.asarray(epoch, jnp.int32) < _TRAIN_AFTER).astype(jnp.float32)
    aux2 = jnp.concatenate(
        [y.astype(jnp.float32), loss_reg, loss_sem, loss_str, flag.reshape(1),
         jnp.zeros((_B - 25,), jnp.float32)]).reshape(2, _G1, _G2)
    gate3 = gate_logits.T.reshape(_E, _G1, _G2)
    el0 = expert_logits[:, :, 0].reshape(_E, _G1, _G2)
    el1 = expert_logits[:, :, 1].reshape(_E, _G1, _G2)

    agg, total = pl.pallas_call(
        _body,
        grid=(_NSTEP,),
        in_specs=[
            pl.BlockSpec((_E, _G1, _G2), lambda i: (0, 0, 0)),
            pl.BlockSpec((_E, _G1, _G2), lambda i: (0, 0, 0)),
            pl.BlockSpec((_E, _G1, _G2), lambda i: (0, 0, 0)),
            pl.BlockSpec((2, _G1, _G2), lambda i: (0, 0, 0)),
            pl.BlockSpec((_E, _NN), lambda i: (0, 0)),
            pl.BlockSpec((_E, _CHUNK), lambda i: (0, i)),
        ],
        out_specs=[
            pl.BlockSpec((_C, _B), lambda i: (0, 0)),
            pl.BlockSpec((1, 1), lambda i: (0, 0)),
        ],
        out_shape=[
            jax.ShapeDtypeStruct((_C, _B), jnp.float32),
            jax.ShapeDtypeStruct((1, 1), jnp.float32),
        ],
        scratch_shapes=[
            pltpu.VMEM((_E, _E), jnp.float32),
            pltpu.VMEM((_E, _G1, _G2), jnp.float32),
            pltpu.VMEM((4, _G1, _G2), jnp.float32),
            pltpu.SMEM((2,), jnp.float32),
        ],
    )(gate3, el0, el1, aux2, node_masks, edge_masks)

    return agg.T, total.reshape(())


# speedup vs baseline: 1.0014x; 1.0014x over previous
import jax
import jax.numpy as jnp
from jax import lax
from jax.experimental import pallas as pl
from jax.experimental.pallas import tpu as pltpu

_E = 8
_B = 4096
_C = 2
_NN = 100000
_NE = 1600000
_TRAIN_AFTER = 10
_ALPHA = 1.5
_W_CE, _W_REG, _W_SEM, _W_STR, _W_DIV, _W_LOAD = 1.0, 0.5, 0.5, 0.5, 0.1, 0.01

_CHUNK = 320000
_NSTEP = _NE // _CHUNK
_ITERS_PER_STEP = 9
_G1, _G2 = 8, 512


def _sqp(z):
    zc = jnp.maximum(z, 0.0)
    return zc * zc


def _eyef(k):
    return (lax.broadcasted_iota(jnp.int32, (k, k), 0)
            == lax.broadcasted_iota(jnp.int32, (k, k), 1)).astype(jnp.float32)


def _offdiag_mean_from_gram(G):
    K = G.shape[0]
    eyef = _eyef(K)
    diag_row = jnp.sum(G * eyef, axis=0, keepdims=True)
    ninv_row = 1.0 / jnp.maximum(jnp.sqrt(diag_row), 1e-12)
    ninv_col = jnp.sum(eyef * ninv_row, axis=1, keepdims=True)
    S = G * ninv_col * ninv_row
    full = jnp.sum(S)
    diag = jnp.sum(S * eyef)
    return (full - diag) / (K * (K - 1))


def _body(gate_ref, el0_ref, el1_ref, aux_ref, node_ref, edge_ref,
          agg_ref, total_ref, acc_ref, xs_ref, st_ref, sm_ref):
    i = pl.program_id(0)

    @pl.when(i == 0)
    def _init_acc():
        acc_ref[:, :] = jnp.zeros((_E, _E), jnp.float32)

    x = edge_ref[:, :]
    acc_ref[:, :] += lax.dot_general(
        x, x, (((1,), (1,)), ((), ())), preferred_element_type=jnp.float32)

    @pl.when(i == 0)
    def _init():
        nm = node_ref[:, :]
        Gn = lax.dot_general(nm, nm, (((1,), (1,)), ((), ())),
                             preferred_element_type=jnp.float32)
        sm_ref[0] = _offdiag_mean_from_gram(Gn)

        gate = gate_ref[:, :, :]
        flag = aux_ref[1:2, 0:1, 24:25]
        uniform = jnp.full((_E, _G1, _G2), 1.0 / _E, jnp.float32)
        gw0 = jnp.where(flag > 0.0, uniform, gate)
        Xs = gw0 * (_ALPHA - 1.0)
        xs_ref[:, :, :] = Xs
        max_val = jnp.max(Xs, axis=0, keepdims=True)
        tau_lo = max_val - 1.0
        tau_hi = max_val - (1.0 / _E) ** (_ALPHA - 1.0)
        f_lo = jnp.sum(_sqp(Xs - tau_lo), axis=0, keepdims=True) - 1.0
        st_ref[0:1] = tau_lo
        st_ref[1:2] = tau_hi - tau_lo
        st_ref[2:3] = tau_lo
        st_ref[3:4] = f_lo

    @pl.when(i > 0)
    def _bisect():
        Xs = xs_ref[:, :, :]
        tau_lo = st_ref[0:1]
        dm = st_ref[1:2]
        f_lo = st_ref[3:4]
        tau_m = tau_lo
        for _ in range(_ITERS_PER_STEP):
            dm = dm / 2.0
            tau_m = tau_lo + dm
            p_m = _sqp(Xs - tau_m)
            f_m = jnp.sum(p_m, axis=0, keepdims=True) - 1.0
            tau_lo = jnp.where((f_m * f_lo) >= 0, tau_m, tau_lo)
        st_ref[0:1] = tau_lo
        st_ref[1:2] = dm
        st_ref[2:3] = tau_m

    @pl.when(i == _NSTEP - 1)
    def _final():
        Xs = xs_ref[:, :, :]
        p_m = _sqp(Xs - st_ref[2:3])
        gw = p_m / jnp.sum(p_m, axis=0, keepdims=True)

        agg0 = jnp.sum(el0_ref[:, :, :] * gw, axis=0, keepdims=True)
        agg1 = jnp.sum(el1_ref[:, :, :] * gw, axis=0, keepdims=True)
        agg_ref[0:1, :] = jnp.reshape(agg0, (1, _B))
        agg_ref[1:2, :] = jnp.reshape(agg1, (1, _B))

        yf = aux_ref[0:1]
        c1 = jnp.sum(yf)
        c0 = jnp.float32(_B) - c1
        c0 = jnp.where(c0 == 0.0, 1.0, c0)
        c1 = jnp.where(c1 == 0.0, 1.0, c1)
        w0 = 1.0 / c0
        w1 = 1.0 / c1
        wsum = w0 + w1
        w0 = w0 / wsum
        w1 = w1 / wsum
        m = jnp.maximum(agg0, agg1)
        lse = m + jnp.log(jnp.exp(agg0 - m) + jnp.exp(agg1 - m))
        logp0 = agg0 - lse
        logp1 = agg1 - lse
        is0 = yf == 0.0
        nll = -jnp.where(is0, logp0, logp1)
        wi = jnp.where(is0, w0, w1)
        ce = jnp.sum(wi * nll) / jnp.sum(wi)

        wf_col = jnp.reshape(gw[:, 0:1, 0:1], (_E, 1))
        wf_row = jnp.sum(_eyef(_E) * wf_col, axis=0, keepdims=True)
        wf3 = jnp.reshape(wf_row, (1, 1, _E))
        reg = jnp.sum(aux_ref[1:2, 0:1, 0:8] * wf3)
        sem = jnp.sum(aux_ref[1:2, 0:1, 8:16] * wf3)
        strv = jnp.sum(aux_ref[1:2, 0:1, 16:24] * wf3)

        s2 = jnp.sum(gw, axis=2, keepdims=True)
        avg = jnp.sum(s2, axis=1, keepdims=True) / jnp.float32(_B)
        u = 1.0 / _E
        load = jnp.sum(u * (jnp.log(jnp.full((_E, 1, 1), u, jnp.float32))
                            - jnp.log(avg + 1e-8))) / _E

        off_edge = _offdiag_mean_from_gram(acc_ref[:, :])
        div = (sm_ref[0] + off_edge) / 2.0
        total = (_W_CE * ce + _W_REG * reg + _W_SEM * sem + _W_STR * strv
                 + _W_DIV * div + _W_LOAD * load)
        total_ref[0:1, 0:1] = jnp.reshape(total, (1, 1))


def kernel(gate_logits, expert_logits, node_masks, edge_masks,
           loss_reg, loss_sem, loss_str, y, epoch):
    flag = (jnp.asarray(epoch, jnp.int32) < _TRAIN_AFTER).astype(jnp.float32)
    aux2 = jnp.concatenate(
        [y.astype(jnp.float32), loss_reg, loss_sem, loss_str, flag.reshape(1),
         jnp.zeros((_B - 25,), jnp.float32)]).reshape(2, _G1, _G2)
    gate3 = gate_logits.T.reshape(_E, _G1, _G2)
    el0 = expert_logits[:, :, 0].reshape(_E, _G1, _G2)
    el1 = expert_logits[:, :, 1].reshape(_E, _G1, _G2)

    agg, total = pl.pallas_call(
        _body,
        grid=(_NSTEP,),
        in_specs=[
            pl.BlockSpec((_E, _G1, _G2), lambda i: (0, 0, 0)),
            pl.BlockSpec((_E, _G1, _G2), lambda i: (0, 0, 0)),
            pl.BlockSpec((_E, _G1, _G2), lambda i: (0, 0, 0)),
            pl.BlockSpec((2, _G1, _G2), lambda i: (0, 0, 0)),
            pl.BlockSpec((_E, _NN), lambda i: (0, 0)),
            pl.BlockSpec((_E, _CHUNK), lambda i: (0, i)),
        ],
        out_specs=[
            pl.BlockSpec((_C, _B), lambda i: (0, 0)),
            pl.BlockSpec((1, 1), lambda i: (0, 0)),
        ],
        out_shape=[
            jax.ShapeDtypeStruct((_C, _B), jnp.float32),
            jax.ShapeDtypeStruct((1, 1), jnp.float32),
        ],
        scratch_shapes=[
            pltpu.VMEM((_E, _E), jnp.float32),
            pltpu.VMEM((_E, _G1, _G2), jnp.float32),
            pltpu.VMEM((4, _G1, _G2), jnp.float32),
            pltpu.SMEM((2,), jnp.float32),
        ],
    )(gate3, el0, el1, aux2, node_masks, edge_masks)

    return agg.T, total.reshape(())


# speedup vs baseline: 1.0407x; 1.0392x over previous
import jax
import jax.numpy as jnp
from jax import lax
from jax.experimental import pallas as pl
from jax.experimental.pallas import tpu as pltpu

_E = 8
_B = 4096
_C = 2
_NN = 100000
_NE = 1600000
_TRAIN_AFTER = 10
_ALPHA = 1.5
_W_CE, _W_REG, _W_SEM, _W_STR, _W_DIV, _W_LOAD = 1.0, 0.5, 0.5, 0.5, 0.1, 0.01

_CHUNK = 320000
_NSTEP = _NE // _CHUNK
_ITERS_PER_STEP = 9
_G1, _G2 = 8, 512


def _sqp(z):
    zc = jnp.maximum(z, 0.0)
    return zc * zc


def _eyef(k):
    return (lax.broadcasted_iota(jnp.int32, (k, k), 0)
            == lax.broadcasted_iota(jnp.int32, (k, k), 1)).astype(jnp.float32)


def _offdiag_mean_from_gram(G):
    K = G.shape[0]
    eyef = _eyef(K)
    diag_row = jnp.sum(G * eyef, axis=0, keepdims=True)
    ninv_row = 1.0 / jnp.maximum(jnp.sqrt(diag_row), 1e-12)
    ninv_col = jnp.sum(eyef * ninv_row, axis=1, keepdims=True)
    S = G * ninv_col * ninv_row
    full = jnp.sum(S)
    diag = jnp.sum(S * eyef)
    return (full - diag) / (K * (K - 1))


def _body(big_ref, node_ref, edge_ref,
          agg_ref, total_ref, acc_ref, xs_ref, st_ref, sm_ref):
    i = pl.program_id(0)

    @pl.when(i == 0)
    def _init_acc():
        acc_ref[:, :] = jnp.zeros((_E, _E), jnp.float32)

    x = edge_ref[:, :]
    acc_ref[:, :] += lax.dot_general(
        x, x, (((1,), (1,)), ((), ())), preferred_element_type=jnp.float32)

    @pl.when(i == 0)
    def _init():
        nm = node_ref[:, :]
        Gn = lax.dot_general(nm, nm, (((1,), (1,)), ((), ())),
                             preferred_element_type=jnp.float32)
        sm_ref[0] = _offdiag_mean_from_gram(Gn)

        gate = big_ref[0:_E]
        flag = big_ref[25:26, 0:1, 24:25]
        uniform = jnp.full((_E, _G1, _G2), 1.0 / _E, jnp.float32)
        gw0 = jnp.where(flag > 0.0, uniform, gate)
        Xs = gw0 * (_ALPHA - 1.0)
        xs_ref[:, :, :] = Xs
        max_val = jnp.max(Xs, axis=0, keepdims=True)
        tau_lo = max_val - 1.0
        tau_hi = max_val - (1.0 / _E) ** (_ALPHA - 1.0)
        f_lo = jnp.sum(_sqp(Xs - tau_lo), axis=0, keepdims=True) - 1.0
        st_ref[0:1] = tau_lo
        st_ref[1:2] = tau_hi - tau_lo
        st_ref[2:3] = tau_lo
        st_ref[3:4] = f_lo

    @pl.when(i > 0)
    def _bisect():
        Xs = xs_ref[:, :, :]
        tau_lo = st_ref[0:1]
        dm = st_ref[1:2]
        f_lo = st_ref[3:4]
        tau_m = tau_lo
        for _ in range(_ITERS_PER_STEP):
            dm = dm / 2.0
            tau_m = tau_lo + dm
            p_m = _sqp(Xs - tau_m)
            f_m = jnp.sum(p_m, axis=0, keepdims=True) - 1.0
            tau_lo = jnp.where((f_m * f_lo) >= 0, tau_m, tau_lo)
        st_ref[0:1] = tau_lo
        st_ref[1:2] = dm
        st_ref[2:3] = tau_m

    @pl.when(i == _NSTEP - 1)
    def _final():
        Xs = xs_ref[:, :, :]
        p_m = _sqp(Xs - st_ref[2:3])
        gw = p_m / jnp.sum(p_m, axis=0, keepdims=True)

        agg0 = jnp.sum(big_ref[8:16] * gw, axis=0, keepdims=True)
        agg1 = jnp.sum(big_ref[16:24] * gw, axis=0, keepdims=True)
        agg_ref[0:1, :] = jnp.reshape(agg0, (1, _B))
        agg_ref[1:2, :] = jnp.reshape(agg1, (1, _B))

        yf = big_ref[24:25]
        c1 = jnp.sum(yf)
        c0 = jnp.float32(_B) - c1
        c0 = jnp.where(c0 == 0.0, 1.0, c0)
        c1 = jnp.where(c1 == 0.0, 1.0, c1)
        w0 = 1.0 / c0
        w1 = 1.0 / c1
        wsum = w0 + w1
        w0 = w0 / wsum
        w1 = w1 / wsum
        m = jnp.maximum(agg0, agg1)
        lse = m + jnp.log(jnp.exp(agg0 - m) + jnp.exp(agg1 - m))
        logp0 = agg0 - lse
        logp1 = agg1 - lse
        is0 = yf == 0.0
        nll = -jnp.where(is0, logp0, logp1)
        wi = jnp.where(is0, w0, w1)
        ce = jnp.sum(wi * nll) / jnp.sum(wi)

        wf_col = jnp.reshape(gw[:, 0:1, 0:1], (_E, 1))
        wf_row = jnp.sum(_eyef(_E) * wf_col, axis=0, keepdims=True)
        wf3 = jnp.reshape(wf_row, (1, 1, _E))
        reg = jnp.sum(big_ref[25:26, 0:1, 0:8] * wf3)
        sem = jnp.sum(big_ref[25:26, 0:1, 8:16] * wf3)
        strv = jnp.sum(big_ref[25:26, 0:1, 16:24] * wf3)

        s2 = jnp.sum(gw, axis=2, keepdims=True)
        avg = jnp.sum(s2, axis=1, keepdims=True) / jnp.float32(_B)
        u = 1.0 / _E
        load = jnp.sum(u * (jnp.log(jnp.full((_E, 1, 1), u, jnp.float32))
                            - jnp.log(avg + 1e-8))) / _E

        off_edge = _offdiag_mean_from_gram(acc_ref[:, :])
        div = (sm_ref[0] + off_edge) / 2.0
        total = (_W_CE * ce + _W_REG * reg + _W_SEM * sem + _W_STR * strv
                 + _W_DIV * div + _W_LOAD * load)
        total_ref[0:1, 0:1] = jnp.reshape(total, (1, 1))


def kernel(gate_logits, expert_logits, node_masks, edge_masks,
           loss_reg, loss_sem, loss_str, y, epoch):
    flag = (jnp.asarray(epoch, jnp.int32) < _TRAIN_AFTER).astype(jnp.float32)
    aux = jnp.concatenate(
        [loss_reg, loss_sem, loss_str, flag.reshape(1),
         jnp.zeros((_B - 25,), jnp.float32)]).reshape(1, _G1, _G2)
    big = jnp.concatenate(
        [gate_logits.T.reshape(_E, _G1, _G2),
         expert_logits[:, :, 0].reshape(_E, _G1, _G2),
         expert_logits[:, :, 1].reshape(_E, _G1, _G2),
         y.astype(jnp.float32).reshape(1, _G1, _G2),
         aux], axis=0)

    agg, total = pl.pallas_call(
        _body,
        grid=(_NSTEP,),
        in_specs=[
            pl.BlockSpec((26, _G1, _G2), lambda i: (0, 0, 0)),
            pl.BlockSpec((_E, _NN), lambda i: (0, 0)),
            pl.BlockSpec((_E, _CHUNK), lambda i: (0, i)),
        ],
        out_specs=[
            pl.BlockSpec((_C, _B), lambda i: (0, 0)),
            pl.BlockSpec((1, 1), lambda i: (0, 0)),
        ],
        out_shape=[
            jax.ShapeDtypeStruct((_C, _B), jnp.float32),
            jax.ShapeDtypeStruct((1, 1), jnp.float32),
        ],
        scratch_shapes=[
            pltpu.VMEM((_E, _E), jnp.float32),
            pltpu.VMEM((_E, _G1, _G2), jnp.float32),
            pltpu.VMEM((4, _G1, _G2), jnp.float32),
            pltpu.SMEM((2,), jnp.float32),
        ],
    )(big, node_masks, edge_masks)

    return agg.T, total.reshape(())
